# Initial kernel scaffold; baseline (speedup 1.0000x reference)
import jax
import jax.numpy as jnp
from jax.experimental import pallas as pl


def kernel(edge_index, edge_weight, user_emb, item_emb, W1, b1, W2, b2):
    raise NotImplementedError("write your pallas kernel here")


# speedup vs baseline: 3.1517x; 3.1517x over previous
import functools

import jax
import jax.numpy as jnp
from jax import lax
from jax.experimental import pallas as pl
from jax.experimental.pallas import tpu as pltpu
from jax.experimental.pallas import tpu_sc as plsc

N = 50000
NP = 50048
E = 800000
D = 32
NS = 16
NC = 2
EPT = E // NS
CH = 128
NFULL = EPT // CH
TAIL = EPT - NFULL * CH
RPT = NP // NS
ZR = 136


def _scale_chunk(rows_v, w_v, n):
    for g in range(n // 16):
        wvec = w_v[pl.ds(g * 16, 16)]
        for j in range(16):
            e = g * 16 + j
            wv = wvec[j]
            rows_v[e, pl.ds(0, 16)] = rows_v[e, pl.ds(0, 16)] * wv
            rows_v[e, pl.ds(16, 16)] = rows_v[e, pl.ds(16, 16)] * wv


def _agg_body(xcat, src_hbm, dst_hbm, w_hbm, out, acc, src_v, dst_v, w_v,
              rows_v, zbuf, sem):
    cid = lax.axis_index("c")
    sid = lax.axis_index("s")
    zero16 = jnp.zeros((16,), jnp.float32)
    zero16i = jnp.zeros((16,), jnp.int32)

    def zrow(r, c):
        zbuf[r, pl.ds(0, 16)] = zero16
        zbuf[r, pl.ds(16, 16)] = zero16
        return c
    lax.fori_loop(0, ZR, zrow, 0)
    zbase = sid * RPT

    def zcopy(i, c):
        pltpu.sync_copy(zbuf, acc.at[pl.ds(zbase + i * ZR, ZR)])
        return c
    lax.fori_loop(0, RPT // ZR, zcopy, 0)
    plsc.subcore_barrier()

    ebase = sid * EPT
    off = cid * N

    def do_chunk():
        for g in range(CH // 16):
            sl = pl.ds(g * 16, 16)
            src_v[sl] = src_v[sl] + off
        pltpu.async_copy(xcat.at[src_v], rows_v, sem).wait()
        _scale_chunk(rows_v, w_v, CH)
        pltpu.sync_copy(rows_v, acc.at[dst_v], add=True)

    def chunk(ci, c):
        eo = ebase + ci * CH
        pltpu.sync_copy(src_hbm.at[pl.ds(eo, CH)], src_v)
        pltpu.sync_copy(dst_hbm.at[pl.ds(eo, CH)], dst_v)
        pltpu.sync_copy(w_hbm.at[pl.ds(eo, CH)], w_v)
        do_chunk()
        return c
    lax.fori_loop(0, NFULL, chunk, 0)

    eo = ebase + NFULL * CH
    for g in range(TAIL // 16, CH // 16):
        sl = pl.ds(g * 16, 16)
        src_v[sl] = zero16i
        dst_v[sl] = zero16i
        w_v[sl] = zero16
    pltpu.sync_copy(src_hbm.at[pl.ds(eo, TAIL)], src_v.at[pl.ds(0, TAIL)])
    pltpu.sync_copy(dst_hbm.at[pl.ds(eo, TAIL)], dst_v.at[pl.ds(0, TAIL)])
    pltpu.sync_copy(w_hbm.at[pl.ds(eo, TAIL)], w_v.at[pl.ds(0, TAIL)])
    do_chunk()

    plsc.subcore_barrier()
    pltpu.sync_copy(acc.at[pl.ds(sid * RPT, RPT)],
                    out.at[pl.ds(cid * NP + sid * RPT, RPT)])


_sc_agg = pl.kernel(
    _agg_body,
    out_type=jax.ShapeDtypeStruct((NC * NP, D), jnp.float32),
    mesh=plsc.VectorSubcoreMesh(core_axis_name="c", subcore_axis_name="s",
                                num_cores=NC, num_subcores=NS),
    scratch_types=[
        pltpu.VMEM_SHARED((NP, D), jnp.float32),
        pltpu.VMEM((CH,), jnp.int32),
        pltpu.VMEM((CH,), jnp.int32),
        pltpu.VMEM((CH,), jnp.float32),
        pltpu.VMEM((CH, D), jnp.float32),
        pltpu.VMEM((ZR, D), jnp.float32),
        pltpu.SemaphoreType.DMA,
    ],
    compiler_params=pltpu.CompilerParams(use_tc_tiling_on_sc=False),
)


B = 2000


def _upd_body(split_out, a_ref, x_ref, w_ref, b_ref, o_ref):
    u0 = (a_ref[0] + x_ref[0]) * 0.5
    u1 = (a_ref[1] + x_ref[1]) * 0.5
    u = jnp.concatenate([u0, u1], axis=1)
    w = w_ref[...]
    y = lax.dot_general(u, w, (((1,), (1,)), ((), ())),
                        preferred_element_type=jnp.float32) + b_ref[...]
    if split_out:
        o_ref[0] = y[:, :D]
        o_ref[1] = y[:, D:]
    else:
        o_ref[...] = y


def _make_update(split_out):
    if split_out:
        out_shape = jax.ShapeDtypeStruct((NC, N, D), jnp.float32)
        out_spec = pl.BlockSpec((NC, B, D), lambda i: (0, i, 0))
    else:
        out_shape = jax.ShapeDtypeStruct((N, 2 * D), jnp.float32)
        out_spec = pl.BlockSpec((B, 2 * D), lambda i: (i, 0))
    return pl.pallas_call(
        functools.partial(_upd_body, split_out),
        grid=(N // B,),
        in_specs=[
            pl.BlockSpec((NC, B, D), lambda i: (0, i, 0)),
            pl.BlockSpec((NC, B, D), lambda i: (0, i, 0)),
            pl.BlockSpec((2 * D, 2 * D), lambda i: (0, 0)),
            pl.BlockSpec((1, 2 * D), lambda i: (0, 0)),
        ],
        out_specs=out_spec,
        out_shape=out_shape,
    )


_upd_split = _make_update(True)
_upd_full = _make_update(False)


def kernel(edge_index, edge_weight, user_emb, item_emb, W1, b1, W2, b2):
    src = edge_index[0].astype(jnp.int32)
    dst = edge_index[1].astype(jnp.int32)
    x = jnp.concatenate([user_emb, item_emb], axis=0)
    xcat = jnp.concatenate([x[:, :D], x[:, D:]], axis=0)

    agg = _sc_agg(xcat, src, dst, edge_weight)
    xs = _upd_split(agg.reshape(NC, NP, D), xcat.reshape(NC, N, D),
                    W1, b1.reshape(1, 2 * D))
    xcat2 = xs.reshape(NC * N, D)
    agg2 = _sc_agg(xcat2, src, dst, edge_weight)
    out = _upd_full(agg2.reshape(NC, NP, D), xcat2.reshape(NC, N, D),
                    W2, b2.reshape(1, 2 * D))
    return out
